# baseline (speedup 1.0000x reference)
import functools

import jax
import jax.numpy as jnp
from jax import lax
from jax.experimental import pallas as pl
from jax.experimental.pallas import tpu as pltpu
from jax.experimental.pallas import tpu_sc as plsc

F32 = jnp.float32
I32 = jnp.int32

NC = 2
NS = 16
L = 16
NW = NC * NS
CHUNK = 64

_SC_MESH = plsc.VectorSubcoreMesh(
    core_axis_name="c", subcore_axis_name="s", num_cores=NC, num_subcores=NS)
_SC_PARAMS = pltpu.CompilerParams(needs_layout_passes=False)


def _make_deg_kernel(n_pad, cpt):
    stripe = n_pad // NS

    @functools.partial(
        pl.kernel, mesh=_SC_MESH, compiler_params=_SC_PARAMS,
        out_type=jax.ShapeDtypeStruct((NC, n_pad), F32),
        scratch_types=[
            pltpu.VMEM((CHUNK,), I32),
            pltpu.VMEM((CHUNK,), F32),
            pltpu.VMEM((stripe,), F32),
            pltpu.VMEM_SHARED((n_pad,), F32),
        ],
    )
    def deg_kernel(c3, w3, degp, cix, wv, zbuf, degs):
        cid = lax.axis_index("c")
        sid = lax.axis_index("s")
        wid = sid * NC + cid

        def zb(i, _):
            zbuf[pl.ds(i * L, L)] = jnp.zeros((L,), F32)
            return 0
        lax.fori_loop(0, stripe // L, zb, 0)
        pltpu.sync_copy(zbuf, degs.at[pl.ds(sid * stripe, stripe)])
        plsc.subcore_barrier()

        def chunk(j, _):
            pltpu.sync_copy(c3.at[wid, j], cix)
            pltpu.sync_copy(w3.at[wid, j], wv)
            pltpu.sync_copy(wv, degs.at[cix], add=True)
            return 0
        lax.fori_loop(0, cpt, chunk, 0)
        plsc.subcore_barrier()
        pltpu.sync_copy(degs.at[pl.ds(sid * stripe, stripe)],
                        degp.at[cid, pl.ds(sid * stripe, stripe)])

    return deg_kernel


def _make_norm_kernel(n_pad, cpt):
    @functools.partial(
        pl.kernel, mesh=_SC_MESH, compiler_params=_SC_PARAMS,
        out_type=jax.ShapeDtypeStruct((NW, cpt, CHUNK), F32),
        scratch_types=[
            pltpu.VMEM((n_pad,), F32),
            pltpu.VMEM((CHUNK,), I32),
            pltpu.VMEM((CHUNK,), I32),
            pltpu.VMEM((CHUNK,), F32),
            pltpu.VMEM((CHUNK,), F32),
        ],
    )
    def norm_kernel(dinv_hbm, r3, c3, w3, norm3, dv, rix, cix, wv, nv):
        cid = lax.axis_index("c")
        sid = lax.axis_index("s")
        wid = sid * NC + cid
        pltpu.sync_copy(dinv_hbm, dv)

        def chunk(j, _):
            pltpu.sync_copy(r3.at[wid, j], rix)
            pltpu.sync_copy(c3.at[wid, j], cix)
            pltpu.sync_copy(w3.at[wid, j], wv)
            for g in range(CHUNK // L):
                rr = rix[pl.ds(g * L, L)]
                cc = cix[pl.ds(g * L, L)]
                ww = wv[pl.ds(g * L, L)]
                dr = plsc.load_gather(dv, [rr])
                dc = plsc.load_gather(dv, [cc])
                nv[pl.ds(g * L, L)] = dr * ww * dc
            pltpu.sync_copy(nv, norm3.at[wid, j])
            return 0
        lax.fori_loop(0, cpt, chunk, 0)

    return norm_kernel


def _make_prop_kernel(n_pad, cpt, feat):
    stripe = n_pad // NS
    zr = 64
    nbuf = 4
    assert cpt % nbuf == 0

    @functools.partial(
        pl.kernel, mesh=_SC_MESH, compiler_params=_SC_PARAMS,
        out_type=jax.ShapeDtypeStruct((NC, n_pad, feat), F32),
        scratch_types=[
            pltpu.VMEM((CHUNK,), I32), pltpu.VMEM((CHUNK,), I32),
            pltpu.VMEM((CHUNK,), I32), pltpu.VMEM((CHUNK,), I32),
            pltpu.VMEM((2 * CHUNK,), I32), pltpu.VMEM((2 * CHUNK,), I32),
            pltpu.VMEM((CHUNK,), F32), pltpu.VMEM((CHUNK,), F32),
            pltpu.VMEM((CHUNK,), F32), pltpu.VMEM((CHUNK,), F32),
            pltpu.VMEM((2 * CHUNK, feat), F32),
            pltpu.VMEM((2 * CHUNK, feat), F32),
            pltpu.VMEM((zr, feat), F32),
            pltpu.VMEM_SHARED((n_pad, feat), F32),
            pltpu.SemaphoreType.DMA, pltpu.SemaphoreType.DMA,
            pltpu.SemaphoreType.DMA, pltpu.SemaphoreType.DMA,
            pltpu.SemaphoreType.DMA, pltpu.SemaphoreType.DMA,
            pltpu.SemaphoreType.DMA, pltpu.SemaphoreType.DMA,
            pltpu.SemaphoreType.DMA, pltpu.SemaphoreType.DMA,
        ],
    )
    def prop_kernel(a_hbm, r3, c3p, norm3, pp,
                    r0, r1, r2, r3v, cp0, cp1,
                    n0, n1, n2, n3,
                    big0, big1, zbuf, acc,
                    is0, is1, is2, is3, gs0, gs1, gs2, gs3, cs0, cs1):
        cid = lax.axis_index("c")
        sid = lax.axis_index("s")
        wid = sid * NC + cid
        ri = (r0, r1, r2, r3v)
        ci2 = (cp0, cp1)
        nv = (n0, n1, n2, n3)
        big = (big0, big1)
        isem = (is0, is1, is2, is3)
        gsem = (gs0, gs1, gs2, gs3)
        csem = (cs0, cs1)

        def fetch_c(m, q):
            pltpu.async_copy(c3p.at[wid, m], ci2[q], csem[q])

        def wait_c(m, q):
            pltpu.make_async_copy(c3p.at[wid, m], ci2[q], csem[q]).wait()

        def idx_fetch(j, b):
            pltpu.async_copy(r3.at[wid, j], ri[b], isem[b])
            pltpu.async_copy(norm3.at[wid, j], nv[b], isem[b])

        def idx_wait(j, b):
            pltpu.make_async_copy(r3.at[wid, j], ri[b], isem[b]).wait()
            pltpu.make_async_copy(norm3.at[wid, j], nv[b], isem[b]).wait()

        def dst_half(b):
            p = (b // 2) % 2
            h = b % 2
            return big[p].at[pl.ds(h * CHUNK, CHUNK)]

        def gather(b):
            pltpu.async_copy(a_hbm.at[ri[b]], dst_half(b), gsem[b])

        def process(b):
            p = (b // 2) % 2
            h = b % 2
            rb = big[p]
            pltpu.make_async_copy(a_hbm.at[ri[b]], dst_half(b), gsem[b]).wait()

            def edge4(q, _):
                e0 = h * CHUNK + q * 4
                nbv = [plsc.load_gather(nv[b], [jnp.full((L,), q * 4 + k, I32)])
                       for k in range(4)]
                for k in range(4):
                    for f in range(feat // L):
                        rb[e0 + k, pl.ds(f * L, L)] = (
                            rb[e0 + k, pl.ds(f * L, L)] * nbv[k])
                return 0
            lax.fori_loop(0, CHUNK // 4, edge4, 0)

        for j in range(3):
            idx_fetch(j, j)
        fetch_c(0, 0)
        fetch_c(1, 1)
        for j in range(3):
            idx_wait(j, j)
            gather(j)
        idx_fetch(3, 3)

        def zrow(i, _):
            def zcol(f, _):
                zbuf[i, pl.ds(f * L, L)] = jnp.zeros((L,), F32)
                return 0
            lax.fori_loop(0, feat // L, zcol, 0)
            return 0
        lax.fori_loop(0, zr, zrow, 0)

        def zc(k, _):
            pltpu.sync_copy(zbuf, acc.at[pl.ds(sid * stripe + k * zr, zr)])
            return 0
        lax.fori_loop(0, stripe // zr, zc, 0)
        plsc.subcore_barrier()

        def group(k, _):
            jb = k * nbuf
            for b in range(nbuf):
                j = jb + b
                bg = (b + 3) % nbuf
                idx_wait(j + 3, bg)
                if b % 2 == 0:
                    gather(bg)
                    process(b)
                else:
                    q = (b - 1) // 2
                    m = (j - 1) // 2
                    process(b)
                    wait_c(m, q)
                    pltpu.sync_copy(big[q], acc.at[ci2[q]], add=True)
                    gather(bg)
                    fetch_c(m + 2, q)
                idx_fetch(j + 4, b)
            return 0
        lax.fori_loop(0, cpt // nbuf - 1, group, 0)

        for j in range(cpt - nbuf, cpt):
            b = j % nbuf
            bg = (b + 3) % nbuf
            if j + 3 < cpt:
                idx_wait(j + 3, bg)
            if b % 2 == 0:
                if j + 3 < cpt:
                    gather(bg)
                process(b)
            else:
                q = (b - 1) // 2
                m = (j - 1) // 2
                process(b)
                wait_c(m, q)
                pltpu.sync_copy(big[q], acc.at[ci2[q]], add=True)
                if j + 3 < cpt:
                    gather(bg)
        plsc.subcore_barrier()
        pltpu.sync_copy(acc.at[pl.ds(sid * stripe, stripe)],
                        pp.at[cid, pl.ds(sid * stripe, stripe)])

    return prop_kernel


def _edge_mlp(ea_p, w1, b1, w2, b2, e_real):
    e_pad = ea_p.shape[0]
    blk = NW * CHUNK
    grid = e_pad // blk

    def body(ea_ref, w1_ref, b1_ref, w2_ref, b2_ref, out_ref):
        i = pl.program_id(0)
        hid = jnp.maximum(
            jnp.dot(ea_ref[...], w1_ref[...], preferred_element_type=F32)
            + b1_ref[...], 0.0)
        ew = jax.nn.sigmoid(
            jnp.dot(hid, w2_ref[...], preferred_element_type=F32) + b2_ref[...])
        rid = i * blk + lax.broadcasted_iota(I32, (blk, 1), 0)
        out_ref[...] = jnp.where(rid < e_real, ew, 0.0)

    return pl.pallas_call(
        body,
        grid=(grid,),
        in_specs=[
            pl.BlockSpec((blk, 16), lambda i: (i, 0)),
            pl.BlockSpec((16, 64), lambda i: (0, 0)),
            pl.BlockSpec((1, 64), lambda i: (0, 0)),
            pl.BlockSpec((64, 1), lambda i: (0, 0)),
            pl.BlockSpec((1, 1), lambda i: (0, 0)),
        ],
        out_specs=pl.BlockSpec((blk, 1), lambda i: (i, 0)),
        out_shape=jax.ShapeDtypeStruct((e_pad, 1), F32),
    )(ea_p, w1, b1.reshape(1, 64), w2, b2.reshape(1, 1))


def _embed_mlp(x_p, w1, b1, w2, b2, rb):
    n_pad = x_p.shape[0]

    def body(x_ref, w1_ref, b1_ref, w2_ref, b2_ref, out_ref):
        h = jnp.maximum(
            jnp.dot(x_ref[...], w1_ref[...], preferred_element_type=F32)
            + b1_ref[...], 0.0)
        h2 = jnp.maximum(
            jnp.dot(h, w2_ref[...], preferred_element_type=F32)
            + b2_ref[...], 0.0)
        out_ref[...] = jnp.concatenate([h2, jnp.zeros_like(h2)], axis=1)

    return pl.pallas_call(
        body,
        grid=(n_pad // rb,),
        in_specs=[
            pl.BlockSpec((rb, 128), lambda i: (i, 0)),
            pl.BlockSpec((128, 64), lambda i: (0, 0)),
            pl.BlockSpec((1, 64), lambda i: (0, 0)),
            pl.BlockSpec((64, 64), lambda i: (0, 0)),
            pl.BlockSpec((1, 64), lambda i: (0, 0)),
        ],
        out_specs=pl.BlockSpec((rb, 128), lambda i: (i, 0)),
        out_shape=jax.ShapeDtypeStruct((n_pad, 128), F32),
    )(x_p, w1, b1.reshape(1, 64), w2, b2.reshape(1, 64))


def _dinv(degp):
    n_pad = degp.shape[1]

    def body(d_ref, dinv_ref, d2_ref):
        deg = d_ref[0, :] + d_ref[1, :] + 1.0
        di = lax.rsqrt(deg)
        dinv_ref[...] = di.reshape(1, n_pad)
        d2_ref[...] = (di * di).reshape(1, n_pad)

    return pl.pallas_call(
        body,
        out_shape=(jax.ShapeDtypeStruct((1, n_pad), F32),
                   jax.ShapeDtypeStruct((1, n_pad), F32)),
    )(degp)


def _layer(pp, a, d2col, w, b, skip, rb):
    n_pad = a.shape[0]
    feat = a.shape[1]
    have_skip = skip is not None

    def body(*refs):
        if have_skip:
            p0_ref, p1_ref, a_ref, d2_ref, w_ref, b_ref, s_ref, out_ref = refs
        else:
            p0_ref, p1_ref, a_ref, d2_ref, w_ref, b_ref, out_ref = refs
        g = p0_ref[...] + p1_ref[...] + a_ref[...] * d2_ref[...]
        h = jnp.dot(g, w_ref[...], preferred_element_type=F32) + b_ref[...]
        act = jnp.maximum(h, 0.0)
        if have_skip:
            act = act + s_ref[...]
        out_ref[...] = act

    in_specs = [
        pl.BlockSpec((rb, feat), lambda i: (i, 0)),
        pl.BlockSpec((rb, feat), lambda i: (i, 0)),
        pl.BlockSpec((rb, feat), lambda i: (i, 0)),
        pl.BlockSpec((rb, 1), lambda i: (i, 0)),
        pl.BlockSpec((feat, 128), lambda i: (0, 0)),
        pl.BlockSpec((1, 128), lambda i: (0, 0)),
    ]
    args = [pp[0], pp[1], a, d2col, w, b.reshape(1, 128)]
    if have_skip:
        in_specs.append(pl.BlockSpec((rb, 128), lambda i: (i, 0)))
        args.append(skip)

    return pl.pallas_call(
        body,
        grid=(n_pad // rb,),
        in_specs=in_specs,
        out_specs=pl.BlockSpec((rb, 128), lambda i: (i, 0)),
        out_shape=jax.ShapeDtypeStruct((n_pad, 128), F32),
    )(*args)


def _head(fin, batch3, fc1_w, fc1_b, head_w, head_b, groups, rb):
    n_pad = fin.shape[0]
    grid = n_pad // rb
    nout = head_w.shape[1]

    def body(f_ref, b_ref, fw_ref, fb_ref, hw_ref, hb_ref, out_ref, sums, cnt):
        i = pl.program_id(0)

        @pl.when(i == 0)
        def _():
            sums[...] = jnp.zeros_like(sums)
            cnt[...] = jnp.zeros_like(cnt)

        oh = (lax.broadcasted_iota(I32, (groups, rb), 0)
              == b_ref[0]).astype(F32)
        sums[...] += jnp.dot(oh, f_ref[...], preferred_element_type=F32)
        cnt[...] += jnp.sum(oh, axis=1, keepdims=True)

        @pl.when(i == grid - 1)
        def _():
            pooled = sums[...] / jnp.maximum(cnt[...], 1.0)
            z = jnp.maximum(
                jnp.dot(pooled, fw_ref[...], preferred_element_type=F32)
                + fb_ref[...], 0.0)
            out_ref[...] = (jnp.dot(z, hw_ref[...], preferred_element_type=F32)
                            + hb_ref[...])

    return pl.pallas_call(
        body,
        grid=(grid,),
        in_specs=[
            pl.BlockSpec((rb, 128), lambda i: (i, 0)),
            pl.BlockSpec((1, 1, rb), lambda i: (i, 0, 0)),
            pl.BlockSpec((128, 256), lambda i: (0, 0)),
            pl.BlockSpec((1, 256), lambda i: (0, 0)),
            pl.BlockSpec((256, nout), lambda i: (0, 0)),
            pl.BlockSpec((1, nout), lambda i: (0, 0)),
        ],
        out_specs=pl.BlockSpec((groups, nout), lambda i: (0, 0)),
        out_shape=jax.ShapeDtypeStruct((groups, nout), F32),
        scratch_shapes=[
            pltpu.VMEM((groups, 128), F32),
            pltpu.VMEM((groups, 1), F32),
        ],
    )(fin, batch3, fc1_w, fc1_b.reshape(1, 256), head_w, head_b.reshape(1, nout))


def kernel(x, edge_index, edge_attr, batch,
           ep_W1, ep_b1, ep_W2, ep_b2,
           emb_W1, emb_b1, emb_W2, emb_b2,
           c1_W, c1_b, c2_W, c2_b, c3_W, c3_b,
           c4_W, c4_b, c5_W, c5_b, c6_W, c6_b,
           fc1_W, fc1_b, head_W, head_b):
    n = x.shape[0]
    e = edge_index.shape[1]
    groups = 16
    rb = 512
    n_pad = ((n + rb - 1) // rb) * rb
    blk = NW * CHUNK
    cpt = ((e + blk - 1) // blk + 3) // 4 * 4
    e_pad = cpt * blk

    x_p = jnp.pad(x, ((0, n_pad - n), (0, 0)))
    r_p = jnp.pad(edge_index[0], (0, e_pad - e))
    c_p = jnp.pad(edge_index[1], (0, e_pad - e))
    r3 = r_p.reshape(NW, cpt, CHUNK)
    c3 = c_p.reshape(NW, cpt, CHUNK)
    c3p = c_p.reshape(NW, cpt // 2, 2 * CHUNK)
    ea_p = jnp.pad(jnp.squeeze(edge_attr, 2), ((0, e_pad - e), (0, 0)))
    batch3 = jnp.pad(batch, (0, n_pad - n), constant_values=-1).reshape(
        n_pad // rb, 1, rb)

    ew = _edge_mlp(ea_p, ep_W1, ep_b1, ep_W2, ep_b2, e)
    w3 = ew.reshape(NW, cpt, CHUNK)

    degp = _make_deg_kernel(n_pad, cpt)(c3, w3)
    dinv_row, d2_row = _dinv(degp)
    dinv_flat = dinv_row.reshape(n_pad)
    d2col = d2_row.reshape(n_pad, 1)

    norm3 = _make_norm_kernel(n_pad, cpt)(dinv_flat, r3, c3, w3)

    h0 = _embed_mlp(x_p, emb_W1, emb_b1, emb_W2, emb_b2, rb)

    prop128 = _make_prop_kernel(n_pad, cpt, 128)
    c1_Wp = jnp.pad(c1_W, ((0, 64), (0, 0)))

    a2 = _layer(prop128(h0, r3, c3p, norm3), h0, d2col, c1_Wp, c1_b, None, rb)
    skip = a2
    a3 = _layer(prop128(a2, r3, c3p, norm3), a2, d2col, c2_W, c2_b, None, rb)
    a4 = _layer(prop128(a3, r3, c3p, norm3), a3, d2col, c3_W, c3_b, skip, rb)
    skip = a4
    a5 = _layer(prop128(a4, r3, c3p, norm3), a4, d2col, c4_W, c4_b, None, rb)
    a6 = _layer(prop128(a5, r3, c3p, norm3), a5, d2col, c5_W, c5_b, None, rb)
    fin = _layer(prop128(a6, r3, c3p, norm3), a6, d2col, c6_W, c6_b, skip, rb)

    return _head(fin, batch3, fc1_W, fc1_b, head_W, head_b, groups, rb)


# speedup vs baseline: 1.0461x; 1.0461x over previous
import functools

import jax
import jax.numpy as jnp
from jax import lax
from jax.experimental import pallas as pl
from jax.experimental.pallas import tpu as pltpu
from jax.experimental.pallas import tpu_sc as plsc

F32 = jnp.float32
I32 = jnp.int32

NC = 2
NS = 16
L = 16
NW = NC * NS
CHUNK = 64

_SC_MESH = plsc.VectorSubcoreMesh(
    core_axis_name="c", subcore_axis_name="s", num_cores=NC, num_subcores=NS)
_SC_PARAMS = pltpu.CompilerParams(needs_layout_passes=False)


def _make_deg_kernel(n_pad, cpt):
    stripe = n_pad // NS

    @functools.partial(
        pl.kernel, mesh=_SC_MESH, compiler_params=_SC_PARAMS,
        out_type=jax.ShapeDtypeStruct((NC, n_pad), F32),
        scratch_types=[
            pltpu.VMEM((CHUNK,), I32),
            pltpu.VMEM((CHUNK,), F32),
            pltpu.VMEM((stripe,), F32),
            pltpu.VMEM_SHARED((n_pad,), F32),
        ],
    )
    def deg_kernel(c3, w3, degp, cix, wv, zbuf, degs):
        cid = lax.axis_index("c")
        sid = lax.axis_index("s")
        wid = sid * NC + cid

        def zb(i, _):
            zbuf[pl.ds(i * L, L)] = jnp.zeros((L,), F32)
            return 0
        lax.fori_loop(0, stripe // L, zb, 0)
        pltpu.sync_copy(zbuf, degs.at[pl.ds(sid * stripe, stripe)])
        plsc.subcore_barrier()

        def chunk(j, _):
            pltpu.sync_copy(c3.at[wid, j], cix)
            pltpu.sync_copy(w3.at[wid, j], wv)
            pltpu.sync_copy(wv, degs.at[cix], add=True)
            return 0
        lax.fori_loop(0, cpt, chunk, 0)
        plsc.subcore_barrier()
        pltpu.sync_copy(degs.at[pl.ds(sid * stripe, stripe)],
                        degp.at[cid, pl.ds(sid * stripe, stripe)])

    return deg_kernel


def _make_norm_kernel(n_pad, cpt):
    @functools.partial(
        pl.kernel, mesh=_SC_MESH, compiler_params=_SC_PARAMS,
        out_type=jax.ShapeDtypeStruct((NW, cpt, CHUNK), F32),
        scratch_types=[
            pltpu.VMEM((n_pad,), F32),
            pltpu.VMEM((CHUNK,), I32),
            pltpu.VMEM((CHUNK,), I32),
            pltpu.VMEM((CHUNK,), F32),
            pltpu.VMEM((CHUNK,), F32),
        ],
    )
    def norm_kernel(dinv_hbm, r3, c3, w3, norm3, dv, rix, cix, wv, nv):
        cid = lax.axis_index("c")
        sid = lax.axis_index("s")
        wid = sid * NC + cid
        pltpu.sync_copy(dinv_hbm, dv)

        def chunk(j, _):
            pltpu.sync_copy(r3.at[wid, j], rix)
            pltpu.sync_copy(c3.at[wid, j], cix)
            pltpu.sync_copy(w3.at[wid, j], wv)
            for g in range(CHUNK // L):
                rr = rix[pl.ds(g * L, L)]
                cc = cix[pl.ds(g * L, L)]
                ww = wv[pl.ds(g * L, L)]
                dr = plsc.load_gather(dv, [rr])
                dc = plsc.load_gather(dv, [cc])
                nv[pl.ds(g * L, L)] = dr * ww * dc
            pltpu.sync_copy(nv, norm3.at[wid, j])
            return 0
        lax.fori_loop(0, cpt, chunk, 0)

    return norm_kernel


def _make_prop_kernel(n_pad, cpt, feat):
    stripe = n_pad // NS
    zr = 64
    nbuf = 4
    assert cpt % nbuf == 0

    @functools.partial(
        pl.kernel, mesh=_SC_MESH, compiler_params=_SC_PARAMS,
        out_type=jax.ShapeDtypeStruct((NC, n_pad, feat), F32),
        scratch_types=[
            pltpu.VMEM((CHUNK,), I32), pltpu.VMEM((CHUNK,), I32),
            pltpu.VMEM((CHUNK,), I32), pltpu.VMEM((CHUNK,), I32),
            pltpu.VMEM((CHUNK,), I32), pltpu.VMEM((CHUNK,), I32),
            pltpu.VMEM((CHUNK,), I32), pltpu.VMEM((CHUNK,), I32),
            pltpu.VMEM((CHUNK,), F32), pltpu.VMEM((CHUNK,), F32),
            pltpu.VMEM((CHUNK,), F32), pltpu.VMEM((CHUNK,), F32),
            pltpu.VMEM((CHUNK, feat), F32), pltpu.VMEM((CHUNK, feat), F32),
            pltpu.VMEM((CHUNK, feat), F32), pltpu.VMEM((CHUNK, feat), F32),
            pltpu.VMEM((zr, feat), F32),
            pltpu.VMEM_SHARED((n_pad, feat), F32),
            pltpu.SemaphoreType.DMA, pltpu.SemaphoreType.DMA,
            pltpu.SemaphoreType.DMA, pltpu.SemaphoreType.DMA,
            pltpu.SemaphoreType.DMA, pltpu.SemaphoreType.DMA,
            pltpu.SemaphoreType.DMA, pltpu.SemaphoreType.DMA,
        ],
    )
    def prop_kernel(a_hbm, r3, c3, norm3, pp,
                    r0, r1, r2, r3v, c0, c1, c2, c3v,
                    n0, n1, n2, n3,
                    rows0, rows1, rows2, rows3, zbuf, acc,
                    is0, is1, is2, is3, gs0, gs1, gs2, gs3):
        cid = lax.axis_index("c")
        sid = lax.axis_index("s")
        wid = sid * NC + cid
        ri = (r0, r1, r2, r3v)
        ci = (c0, c1, c2, c3v)
        nv = (n0, n1, n2, n3)
        rows = (rows0, rows1, rows2, rows3)
        isem = (is0, is1, is2, is3)
        gsem = (gs0, gs1, gs2, gs3)

        def idx_fetch(j, b):
            pltpu.async_copy(r3.at[wid, j], ri[b], isem[b])
            pltpu.async_copy(c3.at[wid, j], ci[b], isem[b])
            pltpu.async_copy(norm3.at[wid, j], nv[b], isem[b])

        def idx_wait(j, b):
            pltpu.make_async_copy(r3.at[wid, j], ri[b], isem[b]).wait()
            pltpu.make_async_copy(c3.at[wid, j], ci[b], isem[b]).wait()
            pltpu.make_async_copy(norm3.at[wid, j], nv[b], isem[b]).wait()

        def gather(b):
            pltpu.async_copy(a_hbm.at[ri[b]], rows[b], gsem[b])

        def process(b):
            rb = rows[b]
            pltpu.make_async_copy(a_hbm.at[ri[b]], rb, gsem[b]).wait()

            def edge4(q, _):
                e0 = q * 4
                nbv = [plsc.load_gather(nv[b], [jnp.full((L,), e0 + k, I32)])
                       for k in range(4)]
                for k in range(4):
                    for f in range(feat // L):
                        rb[e0 + k, pl.ds(f * L, L)] = (
                            rb[e0 + k, pl.ds(f * L, L)] * nbv[k])
                return 0
            lax.fori_loop(0, CHUNK // 4, edge4, 0)
            pltpu.sync_copy(rb, acc.at[ci[b]], add=True)

        for j in range(3):
            idx_fetch(j, j)
        for j in range(3):
            idx_wait(j, j)
            gather(j)
        idx_fetch(3, 3)

        def zrow(i, _):
            def zcol(f, _):
                zbuf[i, pl.ds(f * L, L)] = jnp.zeros((L,), F32)
                return 0
            lax.fori_loop(0, feat // L, zcol, 0)
            return 0
        lax.fori_loop(0, zr, zrow, 0)

        def zc(k, _):
            pltpu.sync_copy(zbuf, acc.at[pl.ds(sid * stripe + k * zr, zr)])
            return 0
        lax.fori_loop(0, stripe // zr, zc, 0)
        plsc.subcore_barrier()

        def group(k, _):
            jb = k * nbuf
            for b in range(nbuf):
                j = jb + b
                bg = (b + 3) % nbuf
                idx_wait(j + 3, bg)
                gather(bg)
                process(b)
                idx_fetch(j + 4, b)
            return 0
        lax.fori_loop(0, cpt // nbuf - 1, group, 0)

        for j in range(cpt - nbuf, cpt):
            b = j % nbuf
            if j + 3 < cpt:
                bg = (b + 3) % nbuf
                idx_wait(j + 3, bg)
                gather(bg)
            process(b)
        plsc.subcore_barrier()
        pltpu.sync_copy(acc.at[pl.ds(sid * stripe, stripe)],
                        pp.at[cid, pl.ds(sid * stripe, stripe)])

    return prop_kernel


def _edge_mlp(ea_p, w1, b1, w2, b2, e_real):
    e_pad = ea_p.shape[0]
    blk = NW * CHUNK
    grid = e_pad // blk

    def body(ea_ref, w1_ref, b1_ref, w2_ref, b2_ref, out_ref):
        i = pl.program_id(0)
        hid = jnp.maximum(
            jnp.dot(ea_ref[...], w1_ref[...], preferred_element_type=F32)
            + b1_ref[...], 0.0)
        ew = jax.nn.sigmoid(
            jnp.dot(hid, w2_ref[...], preferred_element_type=F32) + b2_ref[...])
        rid = i * blk + lax.broadcasted_iota(I32, (blk, 1), 0)
        out_ref[...] = jnp.where(rid < e_real, ew, 0.0)

    return pl.pallas_call(
        body,
        grid=(grid,),
        in_specs=[
            pl.BlockSpec((blk, 16), lambda i: (i, 0)),
            pl.BlockSpec((16, 64), lambda i: (0, 0)),
            pl.BlockSpec((1, 64), lambda i: (0, 0)),
            pl.BlockSpec((64, 1), lambda i: (0, 0)),
            pl.BlockSpec((1, 1), lambda i: (0, 0)),
        ],
        out_specs=pl.BlockSpec((blk, 1), lambda i: (i, 0)),
        out_shape=jax.ShapeDtypeStruct((e_pad, 1), F32),
    )(ea_p, w1, b1.reshape(1, 64), w2, b2.reshape(1, 1))


def _embed_mlp(x_p, w1, b1, w2, b2, rb):
    n_pad = x_p.shape[0]

    def body(x_ref, w1_ref, b1_ref, w2_ref, b2_ref, out_ref):
        h = jnp.maximum(
            jnp.dot(x_ref[...], w1_ref[...], preferred_element_type=F32)
            + b1_ref[...], 0.0)
        h2 = jnp.maximum(
            jnp.dot(h, w2_ref[...], preferred_element_type=F32)
            + b2_ref[...], 0.0)
        out_ref[...] = jnp.concatenate([h2, jnp.zeros_like(h2)], axis=1)

    return pl.pallas_call(
        body,
        grid=(n_pad // rb,),
        in_specs=[
            pl.BlockSpec((rb, 128), lambda i: (i, 0)),
            pl.BlockSpec((128, 64), lambda i: (0, 0)),
            pl.BlockSpec((1, 64), lambda i: (0, 0)),
            pl.BlockSpec((64, 64), lambda i: (0, 0)),
            pl.BlockSpec((1, 64), lambda i: (0, 0)),
        ],
        out_specs=pl.BlockSpec((rb, 128), lambda i: (i, 0)),
        out_shape=jax.ShapeDtypeStruct((n_pad, 128), F32),
    )(x_p, w1, b1.reshape(1, 64), w2, b2.reshape(1, 64))


def _dinv(degp):
    n_pad = degp.shape[1]

    def body(d_ref, dinv_ref, d2_ref):
        deg = d_ref[0, :] + d_ref[1, :] + 1.0
        di = lax.rsqrt(deg)
        dinv_ref[...] = di.reshape(1, n_pad)
        d2_ref[...] = (di * di).reshape(1, n_pad)

    return pl.pallas_call(
        body,
        out_shape=(jax.ShapeDtypeStruct((1, n_pad), F32),
                   jax.ShapeDtypeStruct((1, n_pad), F32)),
    )(degp)


def _layer(pp, a, d2col, w, b, skip, rb):
    n_pad = a.shape[0]
    feat = a.shape[1]
    have_skip = skip is not None

    def body(*refs):
        if have_skip:
            p0_ref, p1_ref, a_ref, d2_ref, w_ref, b_ref, s_ref, out_ref = refs
        else:
            p0_ref, p1_ref, a_ref, d2_ref, w_ref, b_ref, out_ref = refs
        g = p0_ref[...] + p1_ref[...] + a_ref[...] * d2_ref[...]
        h = jnp.dot(g, w_ref[...], preferred_element_type=F32) + b_ref[...]
        act = jnp.maximum(h, 0.0)
        if have_skip:
            act = act + s_ref[...]
        out_ref[...] = act

    in_specs = [
        pl.BlockSpec((rb, feat), lambda i: (i, 0)),
        pl.BlockSpec((rb, feat), lambda i: (i, 0)),
        pl.BlockSpec((rb, feat), lambda i: (i, 0)),
        pl.BlockSpec((rb, 1), lambda i: (i, 0)),
        pl.BlockSpec((feat, 128), lambda i: (0, 0)),
        pl.BlockSpec((1, 128), lambda i: (0, 0)),
    ]
    args = [pp[0], pp[1], a, d2col, w, b.reshape(1, 128)]
    if have_skip:
        in_specs.append(pl.BlockSpec((rb, 128), lambda i: (i, 0)))
        args.append(skip)

    return pl.pallas_call(
        body,
        grid=(n_pad // rb,),
        in_specs=in_specs,
        out_specs=pl.BlockSpec((rb, 128), lambda i: (i, 0)),
        out_shape=jax.ShapeDtypeStruct((n_pad, 128), F32),
    )(*args)


def _head(fin, batch3, fc1_w, fc1_b, head_w, head_b, groups, rb):
    n_pad = fin.shape[0]
    grid = n_pad // rb
    nout = head_w.shape[1]

    def body(f_ref, b_ref, fw_ref, fb_ref, hw_ref, hb_ref, out_ref, sums, cnt):
        i = pl.program_id(0)

        @pl.when(i == 0)
        def _():
            sums[...] = jnp.zeros_like(sums)
            cnt[...] = jnp.zeros_like(cnt)

        oh = (lax.broadcasted_iota(I32, (groups, rb), 0)
              == b_ref[0]).astype(F32)
        sums[...] += jnp.dot(oh, f_ref[...], preferred_element_type=F32)
        cnt[...] += jnp.sum(oh, axis=1, keepdims=True)

        @pl.when(i == grid - 1)
        def _():
            pooled = sums[...] / jnp.maximum(cnt[...], 1.0)
            z = jnp.maximum(
                jnp.dot(pooled, fw_ref[...], preferred_element_type=F32)
                + fb_ref[...], 0.0)
            out_ref[...] = (jnp.dot(z, hw_ref[...], preferred_element_type=F32)
                            + hb_ref[...])

    return pl.pallas_call(
        body,
        grid=(grid,),
        in_specs=[
            pl.BlockSpec((rb, 128), lambda i: (i, 0)),
            pl.BlockSpec((1, 1, rb), lambda i: (i, 0, 0)),
            pl.BlockSpec((128, 256), lambda i: (0, 0)),
            pl.BlockSpec((1, 256), lambda i: (0, 0)),
            pl.BlockSpec((256, nout), lambda i: (0, 0)),
            pl.BlockSpec((1, nout), lambda i: (0, 0)),
        ],
        out_specs=pl.BlockSpec((groups, nout), lambda i: (0, 0)),
        out_shape=jax.ShapeDtypeStruct((groups, nout), F32),
        scratch_shapes=[
            pltpu.VMEM((groups, 128), F32),
            pltpu.VMEM((groups, 1), F32),
        ],
    )(fin, batch3, fc1_w, fc1_b.reshape(1, 256), head_w, head_b.reshape(1, nout))


def kernel(x, edge_index, edge_attr, batch,
           ep_W1, ep_b1, ep_W2, ep_b2,
           emb_W1, emb_b1, emb_W2, emb_b2,
           c1_W, c1_b, c2_W, c2_b, c3_W, c3_b,
           c4_W, c4_b, c5_W, c5_b, c6_W, c6_b,
           fc1_W, fc1_b, head_W, head_b):
    n = x.shape[0]
    e = edge_index.shape[1]
    groups = 16
    rb = 512
    n_pad = ((n + rb - 1) // rb) * rb
    blk = NW * CHUNK
    cpt = ((e + blk - 1) // blk + 3) // 4 * 4
    e_pad = cpt * blk

    x_p = jnp.pad(x, ((0, n_pad - n), (0, 0)))
    r_p = jnp.pad(edge_index[0], (0, e_pad - e))
    c_p = jnp.pad(edge_index[1], (0, e_pad - e))
    r3 = r_p.reshape(NW, cpt, CHUNK)
    c3 = c_p.reshape(NW, cpt, CHUNK)
    ea_p = jnp.pad(jnp.squeeze(edge_attr, 2), ((0, e_pad - e), (0, 0)))
    batch3 = jnp.pad(batch, (0, n_pad - n), constant_values=-1).reshape(
        n_pad // rb, 1, rb)

    ew = _edge_mlp(ea_p, ep_W1, ep_b1, ep_W2, ep_b2, e)
    w3 = ew.reshape(NW, cpt, CHUNK)

    degp = _make_deg_kernel(n_pad, cpt)(c3, w3)
    dinv_row, d2_row = _dinv(degp)
    dinv_flat = dinv_row.reshape(n_pad)
    d2col = d2_row.reshape(n_pad, 1)

    norm3 = _make_norm_kernel(n_pad, cpt)(dinv_flat, r3, c3, w3)

    h0 = _embed_mlp(x_p, emb_W1, emb_b1, emb_W2, emb_b2, rb)

    prop128 = _make_prop_kernel(n_pad, cpt, 128)
    c1_Wp = jnp.pad(c1_W, ((0, 64), (0, 0)))

    a2 = _layer(prop128(h0, r3, c3, norm3), h0, d2col, c1_Wp, c1_b, None, rb)
    skip = a2
    a3 = _layer(prop128(a2, r3, c3, norm3), a2, d2col, c2_W, c2_b, None, rb)
    a4 = _layer(prop128(a3, r3, c3, norm3), a3, d2col, c3_W, c3_b, skip, rb)
    skip = a4
    a5 = _layer(prop128(a4, r3, c3, norm3), a4, d2col, c4_W, c4_b, None, rb)
    a6 = _layer(prop128(a5, r3, c3, norm3), a5, d2col, c5_W, c5_b, None, rb)
    fin = _layer(prop128(a6, r3, c3, norm3), a6, d2col, c6_W, c6_b, skip, rb)

    return _head(fin, batch3, fc1_W, fc1_b, head_W, head_b, groups, rb)
